# baseline (speedup 1.0000x reference)
import jax
import jax.numpy as jnp
from jax import lax
from jax.experimental import pallas as pl
from jax.experimental.pallas import tpu as pltpu
from jax.experimental.pallas import tpu_sc as plsc

_NC = 2
_NS = 16
_NW = _NC * _NS

_B = 4096
_L = 50
_E = 64
_BW = _B // _NW
_GR = 8
_NG = _BW // _GR


def _emb_body(src_tok, tgt_tok, src_tab, tgt_tab, src_out, tgt_out,
              toks, tokt, buf0, buf1, s0, s1, w0, w1):
    wid = lax.axis_index("s") * _NC + lax.axis_index("c")
    b0 = wid * _BW

    pltpu.sync_copy(src_tok.at[pl.ds(b0, _BW)], toks)
    pltpu.sync_copy(tgt_tok.at[pl.ds(b0, _BW)], tokt)

    def fire_group(tab, tokv, g, buf, sem):
        for k in range(_GR):
            pltpu.async_copy(tab.at[tokv.at[g * _GR + k]], buf.at[k], sem)

    def drain_group(tab, tokv, g, buf, sem):
        for k in range(_GR):
            pltpu.make_async_copy(tab.at[tokv.at[g * _GR + k]],
                                  buf.at[k], sem).wait()

    def run_table(tab, out, tokv):
        def out_block(g):
            return out.at[pl.ds(b0 + g * _GR, _GR)]

        @pl.loop(0, _NG, step=2)
        def _pair(g):
            @pl.when(g >= 2)
            def _():
                pltpu.make_async_copy(buf0, out_block(g - 2), w0).wait()

            fire_group(tab, tokv, g, buf0, s0)

            @pl.when(g >= 1)
            def _():
                pltpu.make_async_copy(buf1, out_block(g - 1), w1).wait()

            fire_group(tab, tokv, g + 1, buf1, s1)
            drain_group(tab, tokv, g, buf0, s0)
            pltpu.async_copy(buf0, out_block(g), w0)
            drain_group(tab, tokv, g + 1, buf1, s1)
            pltpu.async_copy(buf1, out_block(g + 1), w1)

        pltpu.make_async_copy(buf0, out_block(_NG - 2), w0).wait()
        pltpu.make_async_copy(buf1, out_block(_NG - 1), w1).wait()

    run_table(src_tab, src_out, toks)
    run_table(tgt_tab, tgt_out, tokt)


@jax.jit
def _emb(src_tok, tgt_tok, src_table, tgt_table):
    mesh = plsc.VectorSubcoreMesh(core_axis_name="c", subcore_axis_name="s")
    out_type = [
        jax.ShapeDtypeStruct((_B, _L, _E), jnp.float32),
        jax.ShapeDtypeStruct((_B, _L, _E), jnp.float32),
    ]
    scratch = [
        pltpu.VMEM((_BW, _L), jnp.int32),
        pltpu.VMEM((_BW, _L), jnp.int32),
        pltpu.VMEM((_GR, _L, _E), jnp.float32),
        pltpu.VMEM((_GR, _L, _E), jnp.float32),
        pltpu.SemaphoreType.DMA,
        pltpu.SemaphoreType.DMA,
        pltpu.SemaphoreType.DMA,
        pltpu.SemaphoreType.DMA,
    ]
    fn = pl.kernel(_emb_body, out_type=out_type, mesh=mesh,
                   scratch_types=scratch,
                   compiler_params=pltpu.CompilerParams(
                       use_tc_tiling_on_sc=False))
    return fn(src_tok, tgt_tok, src_table, tgt_table)


def kernel(src_tokens, tgt_tokens, src_table, tgt_table):
    src_emb, tgt_emb = _emb(src_tokens.astype(jnp.int32),
                            tgt_tokens.astype(jnp.int32),
                            src_table, tgt_table)
    return (src_emb, tgt_emb)
